# Initial kernel scaffold; baseline (speedup 1.0000x reference)
import jax
import jax.numpy as jnp
from jax.experimental import pallas as pl


def kernel(assp_features, prototypes):
    raise NotImplementedError("write your pallas kernel here")


# speedup vs baseline: 2.5524x; 2.5524x over previous
import numpy as np
import jax
import jax.numpy as jnp
from jax.experimental import pallas as pl

_SIZE = 64
_HW = _SIZE * _SIZE
_P = _HW // 4
_K = 18
_C = 256
_B = 16
_JBLK = 1024


def _mask_rows() -> np.ndarray:
    rng = np.random.default_rng(0)
    idx = rng.choice(_HW, _P, replace=False)
    m = np.zeros((_HW,), dtype=np.float32)
    m[idx] = 1.0
    return np.broadcast_to(m[None, :], (8, _HW)).copy()


_MASK8 = _mask_rows()


def _body(a_ref, p_ref, pt_ref, m_ref, o_ref):
    a = a_ref[0]
    protos = p_ref[0]
    protos_t = pt_ref[0]

    a_hi = a.astype(jnp.bfloat16)
    a_lo = (a - a_hi.astype(jnp.float32)).astype(jnp.bfloat16)
    p_hi = protos.astype(jnp.bfloat16)
    p_lo = (protos - p_hi.astype(jnp.float32)).astype(jnp.bfloat16)

    f32 = jnp.float32
    s = (jnp.dot(p_hi, a_hi, preferred_element_type=f32)
         + jnp.dot(p_hi, a_lo, preferred_element_type=f32)
         + jnp.dot(p_lo, a_hi, preferred_element_type=f32))

    norms = jnp.sum(protos * protos, axis=1, keepdims=True)
    d = norms - 2.0 * s

    best_v = d[0:1, :]
    best_i = jnp.zeros((1, _JBLK), dtype=jnp.int32)
    for k in range(1, _K):
        row = d[k:k + 1, :]
        take = row < best_v
        best_v = jnp.where(take, row, best_v)
        best_i = jnp.where(take, jnp.int32(k), best_i)

    iota_k = jax.lax.broadcasted_iota(jnp.int32, (_K, _JBLK), 0)
    onehot = (iota_k == best_i).astype(jnp.bfloat16)
    pt_hi = protos_t.astype(jnp.bfloat16)
    pt_lo = (protos_t - pt_hi.astype(jnp.float32)).astype(jnp.bfloat16)
    sel = (jnp.dot(pt_hi, onehot, preferred_element_type=f32)
           + jnp.dot(pt_lo, onehot, preferred_element_type=f32))

    m = m_ref[0:1, :]
    o_ref[0] = jnp.where(m != 0.0, sel, a)


def kernel(assp_features, prototypes):
    a3 = assp_features.reshape(_B, _C, _HW)
    protos_t = jnp.transpose(prototypes, (0, 2, 1))
    mask8 = jnp.asarray(_MASK8)

    grid = (_B, _HW // _JBLK)
    out = pl.pallas_call(
        _body,
        grid=grid,
        in_specs=[
            pl.BlockSpec((1, _C, _JBLK), lambda b, j: (b, 0, j)),
            pl.BlockSpec((1, _K, _C), lambda b, j: (b, 0, 0)),
            pl.BlockSpec((1, _C, _K), lambda b, j: (b, 0, 0)),
            pl.BlockSpec((8, _JBLK), lambda b, j: (0, j)),
        ],
        out_specs=pl.BlockSpec((1, _C, _JBLK), lambda b, j: (b, 0, j)),
        out_shape=jax.ShapeDtypeStruct((_B, _C, _HW), jnp.float32),
    )(a3, prototypes, protos_t, mask8)
    return out.reshape(_B, _C, _SIZE, _SIZE)


# speedup vs baseline: 2.9359x; 1.1503x over previous
import numpy as np
import jax
import jax.numpy as jnp
from jax.experimental import pallas as pl

_SIZE = 64
_HW = _SIZE * _SIZE
_P = _HW // 4
_K = 18
_C = 256
_B = 16
_JBLK = 2048


def _mask_rows() -> np.ndarray:
    rng = np.random.default_rng(0)
    idx = rng.choice(_HW, _P, replace=False)
    m = np.zeros((_HW,), dtype=np.float32)
    m[idx] = 1.0
    return np.broadcast_to(m[None, :], (8, _HW)).copy()


_MASK8 = _mask_rows()


def _body(a_ref, p_ref, p36_ref, pt36_ref, m_ref, o_ref):
    f32 = jnp.float32
    a = a_ref[0]
    protos = p_ref[0]
    p36 = p36_ref[0]
    pt36 = pt36_ref[0]

    a_hi = a.astype(jnp.bfloat16)
    a_lo = (a - a_hi.astype(f32)).astype(jnp.bfloat16)

    s2 = jnp.dot(p36, a_hi, preferred_element_type=f32)
    s_lo = jnp.dot(p36[0:_K, :], a_lo, preferred_element_type=f32)
    s = s2[0:_K, :] + s2[_K:2 * _K, :] + s_lo

    norms = jnp.sum(protos * protos, axis=1, keepdims=True)
    d = norms - 2.0 * s

    best_v = d[0:1, :]
    best_i = jnp.zeros((1, _JBLK), dtype=jnp.int32)
    for k in range(1, _K):
        row = d[k:k + 1, :]
        take = row < best_v
        best_v = jnp.where(take, row, best_v)
        best_i = jnp.where(take, jnp.int32(k), best_i)

    iota36 = jax.lax.broadcasted_iota(jnp.int32, (2 * _K, _JBLK), 0)
    iota_mod = jnp.where(iota36 >= _K, iota36 - _K, iota36)
    onehot2 = (iota_mod == best_i).astype(jnp.bfloat16)
    sel = jnp.dot(pt36, onehot2, preferred_element_type=f32)

    m = m_ref[0:1, :]
    o_ref[0] = jnp.where(m != 0.0, sel, a)


def kernel(assp_features, prototypes):
    f32 = jnp.float32
    a3 = assp_features.reshape(_B, _C, _HW)
    p_hi = prototypes.astype(jnp.bfloat16)
    p_lo = (prototypes - p_hi.astype(f32)).astype(jnp.bfloat16)
    p36 = jnp.concatenate([p_hi, p_lo], axis=1)
    pt36 = jnp.transpose(p36, (0, 2, 1))
    mask8 = jnp.asarray(_MASK8)

    grid = (_B, _HW // _JBLK)
    out = pl.pallas_call(
        _body,
        grid=grid,
        in_specs=[
            pl.BlockSpec((1, _C, _JBLK), lambda b, j: (b, 0, j)),
            pl.BlockSpec((1, _K, _C), lambda b, j: (b, 0, 0)),
            pl.BlockSpec((1, 2 * _K, _C), lambda b, j: (b, 0, 0)),
            pl.BlockSpec((1, _C, 2 * _K), lambda b, j: (b, 0, 0)),
            pl.BlockSpec((8, _JBLK), lambda b, j: (0, j)),
        ],
        out_specs=pl.BlockSpec((1, _C, _JBLK), lambda b, j: (b, 0, j)),
        out_shape=jax.ShapeDtypeStruct((_B, _C, _HW), jnp.float32),
    )(a3, prototypes, p36, pt36, mask8)
    return out.reshape(_B, _C, _SIZE, _SIZE)


# speedup vs baseline: 3.0181x; 1.0280x over previous
import numpy as np
import jax
import jax.numpy as jnp
from jax.experimental import pallas as pl

_SIZE = 64
_HW = _SIZE * _SIZE
_P = _HW // 4
_K = 18
_C = 256
_B = 16
_JBLK = 2048


def _mask_rows() -> np.ndarray:
    rng = np.random.default_rng(0)
    idx = rng.choice(_HW, _P, replace=False)
    m = np.zeros((_HW,), dtype=np.float32)
    m[idx] = 1.0
    return np.broadcast_to(m[None, :], (8, _HW)).copy()


_MASK8 = _mask_rows()


def _body(a_ref, p_ref, pt_ref, m_ref, o_ref):
    f32 = jnp.float32
    a = a_ref[0]
    protos = p_ref[0]
    protos_t = pt_ref[0]

    s = jnp.dot(protos, a, preferred_element_type=f32)
    norms = jnp.sum(protos * protos, axis=1, keepdims=True)
    d = norms - 2.0 * s

    best_v = d[0:1, :]
    best_i = jnp.zeros((1, _JBLK), dtype=jnp.int32)
    for k in range(1, _K):
        row = d[k:k + 1, :]
        take = row < best_v
        best_v = jnp.where(take, row, best_v)
        best_i = jnp.where(take, jnp.int32(k), best_i)

    iota_k = jax.lax.broadcasted_iota(jnp.int32, (_K, _JBLK), 0)
    onehot = (iota_k == best_i).astype(f32)
    sel = jnp.dot(protos_t, onehot, preferred_element_type=f32)

    m = m_ref[0:1, :]
    o_ref[0] = jnp.where(m != 0.0, sel, a)


def kernel(assp_features, prototypes):
    a3 = assp_features.reshape(_B, _C, _HW)
    protos_t = jnp.transpose(prototypes, (0, 2, 1))
    mask8 = jnp.asarray(_MASK8)

    grid = (_B, _HW // _JBLK)
    out = pl.pallas_call(
        _body,
        grid=grid,
        in_specs=[
            pl.BlockSpec((1, _C, _JBLK), lambda b, j: (b, 0, j)),
            pl.BlockSpec((1, _K, _C), lambda b, j: (b, 0, 0)),
            pl.BlockSpec((1, _C, _K), lambda b, j: (b, 0, 0)),
            pl.BlockSpec((8, _JBLK), lambda b, j: (0, j)),
        ],
        out_specs=pl.BlockSpec((1, _C, _JBLK), lambda b, j: (b, 0, j)),
        out_shape=jax.ShapeDtypeStruct((_B, _C, _HW), jnp.float32),
    )(a3, prototypes, protos_t, mask8)
    return out.reshape(_B, _C, _SIZE, _SIZE)
